# baseline (speedup 1.0000x reference)
import functools

import jax
import jax.numpy as jnp
from jax.experimental import pallas as pl
from jax.experimental.pallas import tpu as pltpu

_EPS = 1e-10
_BIG = 2147483647


def _gmb(u):
    return -jnp.log(-jnp.log(u + _EPS) + _EPS)


def _body(dl_ref, pol_ref, t_hr, t_sbp, t_glu, t_po,
          ud_ref, u_hr, u_sbp, u_glu, u_po, up_ref,
          samples_ref, actions_ref, logp_ref,
          diab_s, tsx, *vs, V, C, N, B):
    groups = [tuple(vs[5 * k + j] for j in range(5)) for k in range(4)]
    i = pl.program_id(0)

    @pl.when(i == 0)
    def _init():
        dl = dl_ref[...]
        zd = dl + _gmb(ud_ref[...])
        s0 = (zd[:, 1:2] > zd[:, 0:1]).astype(jnp.int32)
        diab_s[...] = s0
        m2 = jnp.max(dl)
        lse2 = m2 + jnp.log(jnp.sum(jnp.exp(dl - m2)))
        samples_ref[:, 0:1] = s0
        logp_ref[:, 0:1] = jnp.where(s0 == 1, dl[0, 1], dl[0, 0]) - lse2

        pv = pol_ref[...]
        zp = pv + _gmb(up_ref[...])
        a = jnp.argmax(zp, axis=1).astype(jnp.int32)[:, None]
        actions_ref[...] = a
        mp = jnp.max(pv)
        lsep = mp + jnp.log(jnp.sum(jnp.exp(pv - mp)))
        ia8 = jax.lax.broadcasted_iota(jnp.int32, zp.shape, 1)
        tvp = jnp.sum(jnp.where(ia8 == a, pv, 0.0), axis=1, keepdims=True)
        logp_ref[:, 5:6] = tvp - lsep

        for (amax, aidx, ats, rm, rs) in groups:
            amax[...] = jnp.full(amax.shape, -jnp.inf, jnp.float32)
            aidx[...] = jnp.zeros(aidx.shape, jnp.int32)
            ats[...] = jnp.zeros(ats.shape, jnp.float32)
            rm[...] = jnp.full(rm.shape, -jnp.inf, jnp.float32)
            rs[...] = jnp.zeros(rs.shape, jnp.float32)

    dmask = diab_s[...] == 1
    W = 128
    KPC = C // W
    tables = [(t_hr, u_hr), (t_sbp, u_sbp), (t_glu, u_glu), (t_po, u_po)]
    tail = V - (N - 1) * C
    ktail = (tail + W - 1) // W

    RB = 8

    def _slice_upd(t_ref, u_ref, dm, r0, k, ak, ai, at, partial_cols=None):
        off = k * W
        u = u_ref[pl.ds(r0, RB), pl.ds(off, W)]
        w = -jnp.log(u + _EPS) + _EPS
        tsel = tsx[pl.ds(r0, RB), pl.ds(off, W)]
        z = tsel - jnp.log(w)
        if partial_cols is not None:
            loc = jax.lax.broadcasted_iota(jnp.int32, (1, W), 1)
            z = jnp.where(loc < partial_cols, z, -jnp.inf)
        code = i * KPC + k
        upd = z > ak
        ak = jnp.where(upd, z, ak)
        ai = jnp.where(upd, code, ai)
        at = jnp.where(upd, tsel, at)
        return ak, ai, at

    def _accum(nslices, last):
        for (t_ref, u_ref), (amax, aidx, ats, rm, rs) in zip(
                tables, groups):
            CL0 = nslices * W
            tsx[:, pl.ds(0, CL0)] = jnp.where(
                dmask, t_ref[1:2, pl.ds(0, CL0)], t_ref[0:1, pl.ds(0, CL0)])
            for rb in range(B // RB):
                r0 = rb * RB
                dm = dmask[r0:r0 + RB, :]
                ak = amax[slice(r0, r0 + RB), :]
                ai = aidx[slice(r0, r0 + RB), :]
                at = ats[slice(r0, r0 + RB), :]
                for k in range(nslices):
                    pc = (tail - k * W) if (
                        last and k == nslices - 1 and tail % W) else None
                    ak, ai, at = _slice_upd(t_ref, u_ref, dm, r0, k,
                                            ak, ai, at, partial_cols=pc)
                amax[slice(r0, r0 + RB), :] = ak
                aidx[slice(r0, r0 + RB), :] = ai
                ats[slice(r0, r0 + RB), :] = at
            CL = nslices * W
            t = t_ref[:, pl.ds(0, CL)]
            if last and tail % W:
                l2 = jax.lax.broadcasted_iota(jnp.int32, (2, CL), 1)
                tm = jnp.max(jnp.where(l2 < tail, t, -jnp.inf),
                             axis=1, keepdims=True)
            else:
                tm = jnp.max(t, axis=1, keepdims=True)
            m_old = rm[...]
            m_new = jnp.maximum(m_old, tm)
            se_t = jnp.exp(t - m_new)
            if last and tail % W:
                se_t = jnp.where(l2 < tail, se_t, 0.0)
            se = jnp.sum(se_t, axis=1, keepdims=True)
            rs[...] = rs[...] * jnp.exp(m_old - m_new) + se
            rm[...] = m_new

    @pl.when(i < N - 1)
    def _full():
        _accum(KPC, False)

    @pl.when(i == N - 1)
    def _last():
        _accum(ktail, True)

    @pl.when(i == N - 1)
    def _fin():
        diab_f = diab_s[...]
        lane = jax.lax.broadcasted_iota(jnp.int32, (B, W), 1)
        for v, (amax, aidx, ats, rm, rs) in enumerate(groups):
            z = amax[...]
            iv = aidx[...] * W + lane
            maxv = jnp.max(z, axis=1, keepdims=True)
            at_max = z == maxv
            bj = jnp.min(jnp.where(at_max, iv, _BIG),
                         axis=1, keepdims=True)
            sel_lane = at_max & (iv == bj)
            tv = jnp.sum(jnp.where(sel_lane, ats[...], 0.0),
                         axis=1, keepdims=True)
            samples_ref[:, v + 1:v + 2] = bj
            lse = rm[...] + jnp.log(rs[...])
            lse_sel = jnp.where(diab_f == 1, lse[1, 0], lse[0, 0])
            logp_ref[:, v + 1:v + 2] = tv - lse_sel


def kernel(s0_diab_logits, s0_hr, s0_sysbp, s0_glucose, s0_percoxyg,
           policy_logits, u_diab, u_hr, u_sysbp, u_glucose, u_percoxyg,
           u_policy):
    B, V = u_hr.shape
    A = u_policy.shape[1]
    C = 8192
    N = pl.cdiv(V, C)
    dl = s0_diab_logits.reshape(1, 2)
    pol = policy_logits.reshape(1, A)

    const2 = lambda i: (0, 0)
    tspec = pl.BlockSpec((2, C), lambda i: (0, i))
    uspec = pl.BlockSpec((B, C), lambda i: (0, i))

    scratch = [pltpu.VMEM((B, 1), jnp.int32),
               pltpu.VMEM((B, C), jnp.float32)]
    for _ in range(4):
        scratch += [pltpu.VMEM((B, 128), jnp.float32),
                    pltpu.VMEM((B, 128), jnp.int32),
                    pltpu.VMEM((B, 128), jnp.float32),
                    pltpu.VMEM((2, 1), jnp.float32),
                    pltpu.VMEM((2, 1), jnp.float32)]

    samples, actions2, logp = pl.pallas_call(
        functools.partial(_body, V=V, C=C, N=N, B=B),
        grid=(N,),
        in_specs=[
            pl.BlockSpec((1, 2), const2),
            pl.BlockSpec((1, A), const2),
            tspec, tspec, tspec, tspec,
            pl.BlockSpec((B, 2), const2),
            uspec, uspec, uspec, uspec,
            pl.BlockSpec((B, A), const2),
        ],
        out_specs=[
            pl.BlockSpec((B, 5), const2),
            pl.BlockSpec((B, 1), const2),
            pl.BlockSpec((B, 6), const2),
        ],
        out_shape=[
            jax.ShapeDtypeStruct((B, 5), jnp.int32),
            jax.ShapeDtypeStruct((B, 1), jnp.int32),
            jax.ShapeDtypeStruct((B, 6), jnp.float32),
        ],
        scratch_shapes=scratch,
        compiler_params=pltpu.CompilerParams(
            dimension_semantics=("arbitrary",)),
    )(dl, pol, s0_hr, s0_sysbp, s0_glucose, s0_percoxyg,
      u_diab, u_hr, u_sysbp, u_glucose, u_percoxyg, u_policy)

    return samples, actions2[:, 0], logp


# speedup vs baseline: 1.1089x; 1.1089x over previous
import functools

import jax
import jax.numpy as jnp
from jax.experimental import pallas as pl
from jax.experimental.pallas import tpu as pltpu

_EPS = 1e-10
_BIG = 2147483647


def _gmb(u):
    return -jnp.log(-jnp.log(u + _EPS) + _EPS)


def _body(dl_ref, pol_ref, t_hr, t_sbp, t_glu, t_po,
          ud_ref, u_hr, u_sbp, u_glu, u_po, up_ref,
          samples_ref, actions_ref, logp_ref,
          diab_s, *vs, V, C, N, B):
    groups = [tuple(vs[5 * k + j] for j in range(5)) for k in range(4)]
    i = pl.program_id(0)

    @pl.when(i == 0)
    def _init():
        dl = dl_ref[...]
        zd = dl + _gmb(ud_ref[...])
        s0 = (zd[:, 1:2] > zd[:, 0:1]).astype(jnp.int32)
        diab_s[...] = s0
        m2 = jnp.max(dl)
        lse2 = m2 + jnp.log(jnp.sum(jnp.exp(dl - m2)))
        samples_ref[:, 0:1] = s0
        logp_ref[:, 0:1] = jnp.where(s0 == 1, dl[0, 1], dl[0, 0]) - lse2

        pv = pol_ref[...]
        zp = pv + _gmb(up_ref[...])
        a = jnp.argmax(zp, axis=1).astype(jnp.int32)[:, None]
        actions_ref[...] = a
        mp = jnp.max(pv)
        lsep = mp + jnp.log(jnp.sum(jnp.exp(pv - mp)))
        ia8 = jax.lax.broadcasted_iota(jnp.int32, zp.shape, 1)
        tvp = jnp.sum(jnp.where(ia8 == a, pv, 0.0), axis=1, keepdims=True)
        logp_ref[:, 5:6] = tvp - lsep

        for (amax, aidx, ats, rm, rs) in groups:
            amax[...] = jnp.full(amax.shape, -jnp.inf, jnp.float32)
            aidx[...] = jnp.zeros(aidx.shape, jnp.int32)
            ats[...] = jnp.zeros(ats.shape, jnp.float32)
            rm[...] = jnp.full(rm.shape, -jnp.inf, jnp.float32)
            rs[...] = jnp.zeros(rs.shape, jnp.float32)

    dmask = diab_s[...] == 1
    W = 128
    KPC = C // W
    tables = [(t_hr, u_hr), (t_sbp, u_sbp), (t_glu, u_glu), (t_po, u_po)]
    tail = V - (N - 1) * C
    ktail = (tail + W - 1) // W

    RB = 8

    def _slice_upd(t_ref, u_ref, dm, r0, k, ak, ai, at, partial_cols=None):
        off = k * W
        u = u_ref[pl.ds(r0, RB), pl.ds(off, W)]
        w = -jnp.log(u + _EPS) + _EPS
        tsel = jnp.where(dm, t_ref[1:2, pl.ds(off, W)],
                         t_ref[0:1, pl.ds(off, W)])
        z = tsel - jnp.log(w)
        if partial_cols is not None:
            loc = jax.lax.broadcasted_iota(jnp.int32, (1, W), 1)
            z = jnp.where(loc < partial_cols, z, -jnp.inf)
        code = i * KPC + k
        upd = z > ak
        ak = jnp.where(upd, z, ak)
        ai = jnp.where(upd, code, ai)
        at = jnp.where(upd, tsel, at)
        return ak, ai, at

    def _accum(nslices, last):
        for (t_ref, u_ref), (amax, aidx, ats, rm, rs) in zip(
                tables, groups):
            for rb in range(B // RB):
                r0 = rb * RB
                dm = dmask[r0:r0 + RB, :]
                ak = amax[slice(r0, r0 + RB), :]
                ai = aidx[slice(r0, r0 + RB), :]
                at = ats[slice(r0, r0 + RB), :]
                for k in range(nslices):
                    pc = (tail - k * W) if (
                        last and k == nslices - 1 and tail % W) else None
                    ak, ai, at = _slice_upd(t_ref, u_ref, dm, r0, k,
                                            ak, ai, at, partial_cols=pc)
                amax[slice(r0, r0 + RB), :] = ak
                aidx[slice(r0, r0 + RB), :] = ai
                ats[slice(r0, r0 + RB), :] = at
            CL = nslices * W
            t = t_ref[:, pl.ds(0, CL)]
            if last and tail % W:
                l2 = jax.lax.broadcasted_iota(jnp.int32, (2, CL), 1)
                tm = jnp.max(jnp.where(l2 < tail, t, -jnp.inf),
                             axis=1, keepdims=True)
            else:
                tm = jnp.max(t, axis=1, keepdims=True)
            m_old = rm[...]
            m_new = jnp.maximum(m_old, tm)
            se_t = jnp.exp(t - m_new)
            if last and tail % W:
                se_t = jnp.where(l2 < tail, se_t, 0.0)
            se = jnp.sum(se_t, axis=1, keepdims=True)
            rs[...] = rs[...] * jnp.exp(m_old - m_new) + se
            rm[...] = m_new

    @pl.when(i < N - 1)
    def _full():
        _accum(KPC, False)

    @pl.when(i == N - 1)
    def _last():
        _accum(ktail, True)

    @pl.when(i == N - 1)
    def _fin():
        diab_f = diab_s[...]
        lane = jax.lax.broadcasted_iota(jnp.int32, (B, W), 1)
        for v, (amax, aidx, ats, rm, rs) in enumerate(groups):
            z = amax[...]
            iv = aidx[...] * W + lane
            maxv = jnp.max(z, axis=1, keepdims=True)
            at_max = z == maxv
            bj = jnp.min(jnp.where(at_max, iv, _BIG),
                         axis=1, keepdims=True)
            sel_lane = at_max & (iv == bj)
            tv = jnp.sum(jnp.where(sel_lane, ats[...], 0.0),
                         axis=1, keepdims=True)
            samples_ref[:, v + 1:v + 2] = bj
            lse = rm[...] + jnp.log(rs[...])
            lse_sel = jnp.where(diab_f == 1, lse[1, 0], lse[0, 0])
            logp_ref[:, v + 1:v + 2] = tv - lse_sel


def kernel(s0_diab_logits, s0_hr, s0_sysbp, s0_glucose, s0_percoxyg,
           policy_logits, u_diab, u_hr, u_sysbp, u_glucose, u_percoxyg,
           u_policy):
    B, V = u_hr.shape
    A = u_policy.shape[1]
    C = 8192
    N = pl.cdiv(V, C)
    dl = s0_diab_logits.reshape(1, 2)
    pol = policy_logits.reshape(1, A)

    const2 = lambda i: (0, 0)
    tspec = pl.BlockSpec((2, C), lambda i: (0, i))
    uspec = pl.BlockSpec((B, C), lambda i: (0, i))

    scratch = [pltpu.VMEM((B, 1), jnp.int32)]
    for _ in range(4):
        scratch += [pltpu.VMEM((B, 128), jnp.float32),
                    pltpu.VMEM((B, 128), jnp.int32),
                    pltpu.VMEM((B, 128), jnp.float32),
                    pltpu.VMEM((2, 1), jnp.float32),
                    pltpu.VMEM((2, 1), jnp.float32)]

    samples, actions2, logp = pl.pallas_call(
        functools.partial(_body, V=V, C=C, N=N, B=B),
        grid=(N,),
        in_specs=[
            pl.BlockSpec((1, 2), const2),
            pl.BlockSpec((1, A), const2),
            tspec, tspec, tspec, tspec,
            pl.BlockSpec((B, 2), const2),
            uspec, uspec, uspec, uspec,
            pl.BlockSpec((B, A), const2),
        ],
        out_specs=[
            pl.BlockSpec((B, 5), const2),
            pl.BlockSpec((B, 1), const2),
            pl.BlockSpec((B, 6), const2),
        ],
        out_shape=[
            jax.ShapeDtypeStruct((B, 5), jnp.int32),
            jax.ShapeDtypeStruct((B, 1), jnp.int32),
            jax.ShapeDtypeStruct((B, 6), jnp.float32),
        ],
        scratch_shapes=scratch,
        compiler_params=pltpu.CompilerParams(
            dimension_semantics=("arbitrary",)),
    )(dl, pol, s0_hr, s0_sysbp, s0_glucose, s0_percoxyg,
      u_diab, u_hr, u_sysbp, u_glucose, u_percoxyg, u_policy)

    return samples, actions2[:, 0], logp
